# Initial kernel scaffold; baseline (speedup 1.0000x reference)
import jax
import jax.numpy as jnp
from jax.experimental import pallas as pl


def kernel(cls_logits_0, cls_logits_1, cls_logits_2, cls_logits_3, cls_logits_4, cnt_logits_0, cnt_logits_1, cnt_logits_2, cnt_logits_3, cnt_logits_4, reg_preds_0, reg_preds_1, reg_preds_2, reg_preds_3, reg_preds_4, gt_boxes, classes):
    raise NotImplementedError("write your pallas kernel here")


# speedup vs baseline: 3.3835x; 3.3835x over previous
import functools

import numpy as np
import jax
import jax.numpy as jnp
from jax import lax
from jax.experimental import pallas as pl
from jax.experimental.pallas import tpu as pltpu
from jax.experimental.pallas import tpu_sc as plsc

_STRIDES = [8, 16, 32, 64, 128]
_LIMITS = [[-1, 64], [64, 128], [128, 256], [256, 512], [512, 999999]]
_LEVEL_HW = [(64, 64), (32, 32), (16, 16), (8, 8), (4, 4)]
_B, _M = 8, 64
_NLOC = sum(h * w for h, w in _LEVEL_HW)
_NLOCP = 5504
_NQ = 4
_LW = _NLOCP // _NQ
_NCHUNK = _LW // 16
_BIG = np.float32(99999999.0)


def _build_loc_tables():
    xs, ys, lo, hi, rad = [], [], [], [], []
    for (h, w), s, (llo, lhi) in zip(_LEVEL_HW, _STRIDES, _LIMITS):
        ix = np.arange(w, dtype=np.float32) * s + s // 2
        iy = np.arange(h, dtype=np.float32) * s + s // 2
        xs.append(np.tile(ix, h))
        ys.append(np.repeat(iy, w))
        lo.append(np.full(h * w, llo, np.float32))
        hi.append(np.full(h * w, lhi, np.float32))
        rad.append(np.full(h * w, s * 1.5, np.float32))
    pad = _NLOCP - _NLOC
    out = []
    for arrs, padval in zip((xs, ys, lo, hi, rad), (0.0, 0.0, 1e9, -1e9, 0.0)):
        a = np.concatenate(arrs)
        out.append(np.concatenate([a, np.full(pad, padval, np.float32)]))
    return out


_XS, _YS, _LO, _HI, _RAD = _build_loc_tables()


def _sc_body(xs_ref, ys_ref, lo_ref, hi_ref, rad_ref, boxf_ref, clsb_ref,
             cls_out, cnt_out, reg_out,
             x_v, y_v, lo_v, hi_v, rad_v, boxf_v, clsb_v,
             ocls_v, ocnt_v, orl_v, ort_v, orr_v, orb_v):
    cid = lax.axis_index("c")
    sid = lax.axis_index("s")
    wid = sid * 2 + cid
    b = wid // _NQ
    q = wid % _NQ
    base = q * _LW

    pltpu.sync_copy(xs_ref.at[pl.ds(base, _LW)], x_v)
    pltpu.sync_copy(ys_ref.at[pl.ds(base, _LW)], y_v)
    pltpu.sync_copy(lo_ref.at[pl.ds(base, _LW)], lo_v)
    pltpu.sync_copy(hi_ref.at[pl.ds(base, _LW)], hi_v)
    pltpu.sync_copy(rad_ref.at[pl.ds(base, _LW)], rad_v)
    pltpu.sync_copy(boxf_ref.at[pl.ds(b * 6 * _M * 16, 6 * _M * 16)], boxf_v)
    pltpu.sync_copy(clsb_ref.at[pl.ds(b * _M * 16, _M * 16)], clsb_v)

    big = jnp.full((16,), _BIG, jnp.float32)

    def chunk(i, carry):
        s16 = pl.ds(i * 16, 16)
        xv = x_v[s16]
        yv = y_v[s16]
        lov = lo_v[s16]
        hiv = hi_v[s16]
        radv = rad_v[s16]

        zero = jnp.zeros((16,), jnp.float32)
        init = (jnp.full((16,), 2e8, jnp.float32), zero, zero, zero, zero,
                jnp.zeros((16,), jnp.int32))

        def boxit(m, st):
            best, sl, stt, sr, sb, scl = st
            mb = m * 16
            x0 = boxf_v[pl.ds(mb, 16)]
            y1 = boxf_v[pl.ds(mb + _M * 16, 16)]
            x2 = boxf_v[pl.ds(mb + 2 * _M * 16, 16)]
            y3 = boxf_v[pl.ds(mb + 3 * _M * 16, 16)]
            cx = boxf_v[pl.ds(mb + 4 * _M * 16, 16)]
            cy = boxf_v[pl.ds(mb + 5 * _M * 16, 16)]
            cl = clsb_v[pl.ds(mb, 16)]
            l = xv - x0
            t = yv - y1
            r = x2 - xv
            bb = y3 - yv
            area = (l + r) * (t + bb)
            mn = jnp.minimum(jnp.minimum(l, t), jnp.minimum(r, bb))
            mx = jnp.maximum(jnp.maximum(l, t), jnp.maximum(r, bb))
            dm = jnp.maximum(jnp.abs(xv - cx), jnp.abs(yv - cy))
            mask = (mn > 0.0) & (mx > lov) & (mx <= hiv) & (dm < radv)
            am = jnp.where(mask, area, big)
            take = am < best
            best = jnp.where(take, am, best)
            sl = jnp.where(take, l, sl)
            stt = jnp.where(take, t, stt)
            sr = jnp.where(take, r, sr)
            sb = jnp.where(take, bb, sb)
            scl = jnp.where(take, cl, scl)
            return best, sl, stt, sr, sb, scl

        best, sl, stt, sr, sb, scl = lax.fori_loop(0, _M, boxit, init,
                                                   unroll=4)
        anyp = best < big
        lrmin = jnp.minimum(sl, sr)
        lrmax = jnp.maximum(sl, sr)
        tbmin = jnp.minimum(stt, sb)
        tbmax = jnp.maximum(stt, sb)
        ratio = (lrmin * tbmin) / (lrmax * tbmax + 1e-10)
        s = jnp.maximum(jnp.maximum(ratio, 0.0), 1e-30)
        bits = lax.bitcast_convert_type(s, jnp.int32)
        yv0 = lax.bitcast_convert_type(jnp.int32(0x5F3759DF) - (bits >> 1),
                                       jnp.float32)
        for _ in range(3):
            yv0 = yv0 * (1.5 - 0.5 * s * yv0 * yv0)
        sq = s * yv0
        neg1 = jnp.full((16,), -1.0, jnp.float32)
        ocnt_v[s16] = jnp.where(anyp, sq, neg1)
        ocls_v[s16] = jnp.where(anyp, scl, jnp.zeros((16,), jnp.int32))
        orl_v[s16] = jnp.where(anyp, sl, neg1)
        ort_v[s16] = jnp.where(anyp, stt, neg1)
        orr_v[s16] = jnp.where(anyp, sr, neg1)
        orb_v[s16] = jnp.where(anyp, sb, neg1)
        return carry

    lax.fori_loop(0, _NCHUNK, chunk, 0)

    obase = b * _NLOCP + base
    rbase = b * 4 * _NLOCP + base
    pltpu.sync_copy(ocls_v, cls_out.at[pl.ds(obase, _LW)])
    pltpu.sync_copy(ocnt_v, cnt_out.at[pl.ds(obase, _LW)])
    pltpu.sync_copy(orl_v, reg_out.at[pl.ds(rbase, _LW)])
    pltpu.sync_copy(ort_v, reg_out.at[pl.ds(rbase + _NLOCP, _LW)])
    pltpu.sync_copy(orr_v, reg_out.at[pl.ds(rbase + 2 * _NLOCP, _LW)])
    pltpu.sync_copy(orb_v, reg_out.at[pl.ds(rbase + 3 * _NLOCP, _LW)])


@functools.partial(
    pl.kernel,
    out_type=(
        jax.ShapeDtypeStruct((_B * _NLOCP,), jnp.int32),
        jax.ShapeDtypeStruct((_B * _NLOCP,), jnp.float32),
        jax.ShapeDtypeStruct((_B * 4 * _NLOCP,), jnp.float32),
    ),
    mesh=plsc.VectorSubcoreMesh(core_axis_name="c", subcore_axis_name="s",
                                num_cores=2, num_subcores=16),
    scratch_types=[
        pltpu.VMEM((_LW,), jnp.float32),
        pltpu.VMEM((_LW,), jnp.float32),
        pltpu.VMEM((_LW,), jnp.float32),
        pltpu.VMEM((_LW,), jnp.float32),
        pltpu.VMEM((_LW,), jnp.float32),
        pltpu.VMEM((6 * _M * 16,), jnp.float32),
        pltpu.VMEM((_M * 16,), jnp.int32),
        pltpu.VMEM((_LW,), jnp.int32),
        pltpu.VMEM((_LW,), jnp.float32),
        pltpu.VMEM((_LW,), jnp.float32),
        pltpu.VMEM((_LW,), jnp.float32),
        pltpu.VMEM((_LW,), jnp.float32),
        pltpu.VMEM((_LW,), jnp.float32),
    ],
)
def _gen_targets_sc(xs, ys, lo, hi, rad, boxf, clsb, cls_out, cnt_out,
                    reg_out, *scratch):
    _sc_body(xs, ys, lo, hi, rad, boxf, clsb, cls_out, cnt_out, reg_out,
             *scratch)


def kernel(cls_logits_0, cls_logits_1, cls_logits_2, cls_logits_3,
           cls_logits_4, cnt_logits_0, cnt_logits_1, cnt_logits_2,
           cnt_logits_3, cnt_logits_4, reg_preds_0, reg_preds_1, reg_preds_2,
           reg_preds_3, reg_preds_4, gt_boxes, classes):
    x0 = gt_boxes[..., 0]
    y1 = gt_boxes[..., 1]
    x2 = gt_boxes[..., 2]
    y3 = gt_boxes[..., 3]
    cx = (x0 + x2) / 2
    cy = (y1 + y3) / 2
    boxf = jnp.stack([x0, y1, x2, y3, cx, cy], axis=1)
    boxf = jnp.broadcast_to(boxf[..., None],
                            (_B, 6, _M, 16)).reshape(_B * 6 * _M * 16)
    clsb = jnp.broadcast_to(classes[..., None],
                            (_B, _M, 16)).reshape(_B * _M * 16)

    cls_flat, cnt_flat, reg_flat = _gen_targets_sc(
        jnp.asarray(_XS), jnp.asarray(_YS), jnp.asarray(_LO),
        jnp.asarray(_HI), jnp.asarray(_RAD), boxf, clsb)

    cls_pad = cls_flat.reshape(_B, _NLOCP)
    cnt_pad = cnt_flat.reshape(_B, _NLOCP)
    reg_pad = reg_flat.reshape(_B, 4, _NLOCP)
    cls_t = cls_pad[:, :_NLOC, None]
    cnt_t = cnt_pad[:, :_NLOC, None]
    reg_t = jnp.transpose(reg_pad, (0, 2, 1))[:, :_NLOC, :]
    return cls_t, cnt_t, reg_t
